# baseline (speedup 1.0000x reference)
import functools

import jax
import jax.numpy as jnp
from jax import lax
from jax.experimental import pallas as pl
from jax.experimental.pallas import tpu as pltpu
from jax.experimental.pallas import tpu_sc as plsc

_NC = 2
_NS = 16
_NW = _NC * _NS

_BLK = 1024


def _stage_a_body(final, total_rows, *refs):
    if final:
        (x_ref, w_ref, ai_ref, ci_ref,
         idx_ref, loss_ref, perp_ref, acc_ref, cnt_ref, nw_ref, w2_ref) = refs
    else:
        (x_ref, w_ref,
         idx_ref, ao_ref, co_ref, acc_ref, cnt_ref, nw_ref, w2_ref) = refs
    i = pl.program_id(0)
    n = pl.num_programs(0)
    blk = x_ref.shape[2]
    k = w_ref.shape[0]
    nch = k // 128

    @pl.when(i == 0)
    def _():
        if final:
            acc_ref[...] = ai_ref[...]
            cnt_ref[...] = ci_ref[...]
        else:
            acc_ref[...] = jnp.zeros_like(acc_ref)
            cnt_ref[...] = jnp.zeros_like(cnt_ref)
        w = w_ref[...]
        nw_ref[...] = -(w + w)
        o8 = jnp.ones((8, w.shape[1]), jnp.float32)
        w2_ref[...] = lax.dot_general(w * w, o8, (((1,), (1,)), ((), ())),
                                      preferred_element_type=jnp.float32)

    xt = x_ref[0]
    ones8 = jnp.ones((8, xt.shape[0]), jnp.float32)
    x2row = lax.dot_general(ones8, xt * xt, (((1,), (0,)), ((), ())),
                            preferred_element_type=jnp.float32)[0:1]
    w2 = w2_ref[:, 0:1]
    ji = lax.broadcasted_iota(jnp.int32, (128, blk), 0).astype(jnp.float32)

    minval = None
    minidx = None
    for kb in range(nch):
        nwk = nw_ref[kb * 128:(kb + 1) * 128, :]
        mm2k = lax.dot_general(nwk, xt, (((1,), (0,)), ((), ())),
                               preferred_element_type=jnp.float32)
        dk = (x2row + w2[kb * 128:(kb + 1) * 128]) + mm2k
        if kb == 0:
            minval = dk
            minidx = ji
        else:
            better = dk < minval
            minval = jnp.where(better, dk, minval)
            minidx = jnp.where(better, ji + float(kb * 128), minidx)

    m = jnp.min(minval, axis=0, keepdims=True)
    cand = jnp.where(minval == m, minidx, jnp.float32(2.0 ** 30))
    idxf = jnp.min(cand, axis=0, keepdims=True)
    idx_ref[...] = idxf.astype(jnp.int32).reshape(idx_ref.shape)

    acc_ref[...] += jnp.sum(m, keepdims=True)
    rowsc = lax.broadcasted_iota(jnp.int32, (k, blk), 0).astype(jnp.float32)
    onehot = jnp.where(rowsc == idxf, 1.0, 0.0).astype(jnp.float32)
    ones = jnp.ones((8, blk), jnp.float32)
    cnt_ref[...] += lax.dot_general(ones, onehot, (((1,), (1,)), ((), ())),
                                    preferred_element_type=jnp.float32)

    @pl.when(i == n - 1)
    def _():
        if final:
            denom = total_rows * x_ref.shape[1]
            loss_ref[...] = acc_ref[...] * (0.25 / denom)
            p = cnt_ref[0:1, :] * (1.0 / total_rows)
            s = jnp.sum(p * jnp.log(p + 1e-10), keepdims=True)
            perp_ref[...] = jnp.exp(-s)
        else:
            ao_ref[...] = acc_ref[...]
            co_ref[...] = cnt_ref[...]


def _stage_a(x, w, final=False, partials=()):
    nb, dd, t = x.shape
    rows = nb * t
    k = w.shape[0]
    half = nb // 2
    hrows = half * t
    grid = hrows // _BLK
    base = 0 if not final else half

    scratch = [
        pltpu.VMEM((1, 1), jnp.float32),
        pltpu.VMEM((8, k), jnp.float32),
        pltpu.VMEM((k, dd), jnp.float32),
        pltpu.VMEM((k, 8), jnp.float32),
    ]
    in_specs = [
        pl.BlockSpec((1, dd, _BLK), lambda i: (i + base, 0, 0)),
        pl.BlockSpec((k, dd), lambda i: (0, 0)),
    ]
    if final:
        in_specs += [
            pl.BlockSpec((1, 1), lambda i: (0, 0)),
            pl.BlockSpec((8, k), lambda i: (0, 0)),
        ]
        out_specs = [
            pl.BlockSpec((_BLK,), lambda i: (i,)),
            pl.BlockSpec((1, 1), lambda i: (0, 0)),
            pl.BlockSpec((1, 1), lambda i: (0, 0)),
        ]
        out_shape = [
            jax.ShapeDtypeStruct((hrows,), jnp.int32),
            jax.ShapeDtypeStruct((1, 1), jnp.float32),
            jax.ShapeDtypeStruct((1, 1), jnp.float32),
        ]
        args = (x, w) + partials
    else:
        out_specs = [
            pl.BlockSpec((_BLK,), lambda i: (i,)),
            pl.BlockSpec((1, 1), lambda i: (0, 0)),
            pl.BlockSpec((8, k), lambda i: (0, 0)),
        ]
        out_shape = [
            jax.ShapeDtypeStruct((hrows,), jnp.int32),
            jax.ShapeDtypeStruct((1, 1), jnp.float32),
            jax.ShapeDtypeStruct((8, k), jnp.float32),
        ]
        args = (x, w)
    return pl.pallas_call(
        functools.partial(_stage_a_body, final, rows),
        grid=(grid,),
        in_specs=in_specs,
        out_specs=out_specs,
        out_shape=out_shape,
        scratch_shapes=scratch,
    )(*args)


def _sc_gather_body(w_hbm, idx_hbm, q_hbm, idx_v, rows_v, sem):
    cid = lax.axis_index("c")
    sid = lax.axis_index("s")
    wid = sid * _NC + cid
    b = idx_v.shape[0]
    base = wid * b
    pltpu.sync_copy(idx_hbm.at[pl.ds(base, b)], idx_v)
    pltpu.async_copy(w_hbm.at[idx_v], rows_v, sem).wait()
    pltpu.sync_copy(rows_v, q_hbm.at[pl.ds(base, b)])


def _sc_gather(w_pad, idx):
    rows = idx.shape[0]
    k, dpad = w_pad.shape
    b = rows // _NW
    mesh = plsc.VectorSubcoreMesh(core_axis_name="c", subcore_axis_name="s")
    fn = functools.partial(
        pl.kernel,
        mesh=mesh,
        out_type=jax.ShapeDtypeStruct((rows, dpad), jnp.float32),
        scratch_types=[
            pltpu.VMEM((b,), jnp.int32),
            pltpu.VMEM((b, dpad), jnp.float32),
            pltpu.SemaphoreType.DMA,
        ],
    )(_sc_gather_body)
    return fn(w_pad, idx)


def kernel(latents, W):
    orig_shape = latents.shape
    d = orig_shape[-1]

    xt = jnp.swapaxes(latents, 1, 2)
    w_pad = jnp.pad(W, ((0, 0), (0, 128 - d)))
    idx0, acc0, cnt0 = _stage_a(xt, W)
    q0 = _sc_gather(w_pad, idx0)
    idx1, loss, perp = _stage_a(xt, W, final=True, partials=(acc0, cnt0))
    q1 = _sc_gather(w_pad, idx1)
    q = jnp.concatenate([q0, q1], axis=0)
    return (q[:, :d].reshape(orig_shape), loss.reshape(()), perp.reshape(()))


# speedup vs baseline: 1.1710x; 1.1710x over previous
import functools

import jax
import jax.numpy as jnp
from jax import lax
from jax.experimental import pallas as pl
from jax.experimental.pallas import tpu as pltpu
from jax.experimental.pallas import tpu_sc as plsc

_NC = 2
_NS = 16
_NW = _NC * _NS

_BLK = 1024


def _stage_a_body(x_ref, w_ref, idx_ref, loss_ref, perp_ref, acc_ref, cnt_ref,
                  nw_ref, w2_ref):
    i = pl.program_id(0)
    n = pl.num_programs(0)
    blk = x_ref.shape[2]
    k = w_ref.shape[0]
    nch = k // 128

    @pl.when(i == 0)
    def _():
        acc_ref[...] = jnp.zeros_like(acc_ref)
        cnt_ref[...] = jnp.zeros_like(cnt_ref)
        w = w_ref[...]
        nw_ref[...] = -(w + w)
        o8 = jnp.ones((8, w.shape[1]), jnp.float32)
        w2_ref[...] = lax.dot_general(w * w, o8, (((1,), (1,)), ((), ())),
                                      preferred_element_type=jnp.float32)

    xt = x_ref[0]
    ones8 = jnp.ones((8, xt.shape[0]), jnp.float32)
    x2row = lax.dot_general(ones8, xt * xt, (((1,), (0,)), ((), ())),
                            preferred_element_type=jnp.float32)[0:1]
    w2 = w2_ref[:, 0:1]
    ji = lax.broadcasted_iota(jnp.int32, (128, blk), 0).astype(jnp.float32)

    minval = None
    minidx = None
    for kb in range(nch):
        nwk = nw_ref[kb * 128:(kb + 1) * 128, :]
        mm2k = lax.dot_general(nwk, xt, (((1,), (0,)), ((), ())),
                               preferred_element_type=jnp.float32)
        dk = (x2row + w2[kb * 128:(kb + 1) * 128]) + mm2k
        if kb == 0:
            minval = dk
            minidx = ji
        else:
            better = dk < minval
            minval = jnp.where(better, dk, minval)
            minidx = jnp.where(better, ji + float(kb * 128), minidx)

    m = jnp.min(minval, axis=0, keepdims=True)
    cand = jnp.where(minval == m, minidx, jnp.float32(2.0 ** 30))
    idxf = jnp.min(cand, axis=0, keepdims=True)
    idx_ref[...] = idxf.astype(jnp.int32).reshape(idx_ref.shape)

    acc_ref[...] += jnp.sum(m, keepdims=True)
    rowsc = lax.broadcasted_iota(jnp.int32, (k, blk), 0).astype(jnp.float32)
    onehot = jnp.where(rowsc == idxf, 1.0, 0.0).astype(jnp.float32)
    ones = jnp.ones((8, blk), jnp.float32)
    cnt_ref[...] += lax.dot_general(ones, onehot, (((1,), (1,)), ((), ())),
                                    preferred_element_type=jnp.float32)

    @pl.when(i == n - 1)
    def _():
        rows_total = n * blk
        denom = rows_total * x_ref.shape[1]
        loss_ref[...] = acc_ref[...] * (0.25 / denom)
        p = cnt_ref[0:1, :] * (1.0 / rows_total)
        s = jnp.sum(p * jnp.log(p + 1e-10), keepdims=True)
        perp_ref[...] = jnp.exp(-s)


def _stage_a(x, w):
    nb, dd, t = x.shape
    rows = nb * t
    k = w.shape[0]
    grid = rows // _BLK
    return pl.pallas_call(
        _stage_a_body,
        grid=(grid,),
        in_specs=[
            pl.BlockSpec((1, dd, _BLK), lambda i: (i, 0, 0)),
            pl.BlockSpec((k, dd), lambda i: (0, 0)),
        ],
        out_specs=[
            pl.BlockSpec((_BLK,), lambda i: (i,)),
            pl.BlockSpec((1, 1), lambda i: (0, 0)),
            pl.BlockSpec((1, 1), lambda i: (0, 0)),
        ],
        out_shape=[
            jax.ShapeDtypeStruct((rows,), jnp.int32),
            jax.ShapeDtypeStruct((1, 1), jnp.float32),
            jax.ShapeDtypeStruct((1, 1), jnp.float32),
        ],
        scratch_shapes=[
            pltpu.VMEM((1, 1), jnp.float32),
            pltpu.VMEM((8, k), jnp.float32),
            pltpu.VMEM((k, dd), jnp.float32),
            pltpu.VMEM((k, 8), jnp.float32),
        ],
    )(x, w)


def _sc_gather_body(w_hbm, idx_hbm, q_hbm, idx_v, rows_v, sem):
    cid = lax.axis_index("c")
    sid = lax.axis_index("s")
    wid = sid * _NC + cid
    b = idx_v.shape[0]
    base = wid * b
    pltpu.sync_copy(idx_hbm.at[pl.ds(base, b)], idx_v)
    pltpu.async_copy(w_hbm.at[idx_v], rows_v, sem).wait()
    pltpu.sync_copy(rows_v, q_hbm.at[pl.ds(base, b)])


def _sc_gather(w_pad, idx):
    rows = idx.shape[0]
    k, dpad = w_pad.shape
    b = rows // _NW
    mesh = plsc.VectorSubcoreMesh(core_axis_name="c", subcore_axis_name="s")
    fn = functools.partial(
        pl.kernel,
        mesh=mesh,
        out_type=jax.ShapeDtypeStruct((rows, dpad), jnp.float32),
        scratch_types=[
            pltpu.VMEM((b,), jnp.int32),
            pltpu.VMEM((b, dpad), jnp.float32),
            pltpu.SemaphoreType.DMA,
        ],
    )(_sc_gather_body)
    return fn(w_pad, idx)


def kernel(latents, W):
    orig_shape = latents.shape
    d = orig_shape[-1]

    xt = jnp.swapaxes(latents, 1, 2)
    idx, loss, perp = _stage_a(xt, W)
    w_pad = jnp.pad(W, ((0, 0), (0, 128 - d)))
    q = _sc_gather(w_pad, idx)
    return (q[:, :d].reshape(orig_shape), loss.reshape(()), perp.reshape(()))
